# baseline (speedup 1.0000x reference)
import functools

import jax
import jax.numpy as jnp
from jax.experimental import pallas as pl
from jax.experimental.pallas import tpu as pltpu

_KP = 64
_INF = float("inf")


def _triplet_block(a_ref, f_ref, lr_ref, lc_ref, m_ref, o_ref, ah_s, b_s, acc,
                   *, nsteps, ba):
    i = pl.program_id(0)
    n, d = f_ref.shape

    @pl.when(i == 0)
    def _():
        f = f_ref[...]
        fsq = jnp.sum(f * f, axis=1, keepdims=True)
        ones = jnp.ones((n, 1), jnp.float32)
        pad = jnp.zeros((n, _KP - (d + 1)), jnp.float32)
        b_s[...] = jnp.concatenate([f, fsq, pad], axis=1)
        ah_s[...] = jnp.concatenate([-2.0 * f, ones, pad], axis=1)

    dims = (((1,), (1,)), ((), ()))
    b_all = b_s[...]
    lc = lc_ref[...]
    ch = 1024
    pos_parts, neg_parts = [], []
    for k in range(ba // ch):
        rows = pl.ds(i * ba + k * ch, ch)
        t = jax.lax.dot_general(ah_s[rows, :], b_all, dims,
                                preferred_element_type=jnp.float32)
        pm = lr_ref[rows, :] == lc
        pos_parts.append(
            jnp.max(jnp.where(pm, t, -_INF), axis=1, keepdims=True))
        neg_parts.append(
            jnp.min(jnp.where(pm, _INF, t), axis=1, keepdims=True))
    pos_t = jnp.concatenate(pos_parts, axis=0)
    neg_t = jnp.concatenate(neg_parts, axis=0)
    valid = neg_t < _INF

    a = a_ref[...]
    asq = jnp.sum(a * a, axis=1, keepdims=True)
    pos_d2 = jnp.maximum(pos_t + asq, 0.0)
    neg_d2 = jnp.maximum(jnp.where(valid, neg_t, 0.0) + asq, 0.0)

    margin = m_ref[0, 0]
    per = jnp.maximum(jnp.sqrt(pos_d2) - jnp.sqrt(neg_d2) + margin, 0.0)
    per = jnp.where(valid, per, 0.0)

    s = jnp.sum(per, axis=0, keepdims=True)[0, 0]
    c = jnp.sum(valid.astype(jnp.float32), axis=0, keepdims=True)[0, 0]
    tot_s = jnp.where(i == 0, 0.0, acc[0, 0]) + s
    tot_c = jnp.where(i == 0, 0.0, acc[1, 0]) + c
    acc[0, 0] = tot_s
    acc[1, 0] = tot_c

    @pl.when(i == nsteps - 1)
    def _():
        loss = jnp.where(tot_c > 0.0, tot_s / jnp.maximum(tot_c, 1.0), 0.0)
        o_ref[...] = jnp.full((1, 1), loss, jnp.float32)


def kernel(features, labels, margin):
    n, d = features.shape
    ba = 4096
    nsteps = n // ba
    labels_col = labels.reshape(n, 1).astype(jnp.int32)
    labels_row = labels.reshape(1, n).astype(jnp.int32)
    margin_arr = jnp.asarray(margin, jnp.float32).reshape(1, 1)

    out = pl.pallas_call(
        functools.partial(_triplet_block, nsteps=nsteps, ba=ba),
        grid=(nsteps,),
        in_specs=[
            pl.BlockSpec((ba, d), lambda i: (i, 0)),
            pl.BlockSpec((n, d), lambda i: (0, 0)),
            pl.BlockSpec((n, 1), lambda i: (0, 0)),
            pl.BlockSpec((1, n), lambda i: (0, 0)),
            pl.BlockSpec((1, 1), lambda i: (0, 0)),
        ],
        out_specs=pl.BlockSpec((1, 1), lambda i: (0, 0)),
        out_shape=jax.ShapeDtypeStruct((1, 1), jnp.float32),
        scratch_shapes=[pltpu.VMEM((n, _KP), jnp.float32),
                        pltpu.VMEM((n, _KP), jnp.float32),
                        pltpu.SMEM((2, 1), jnp.float32)],
    )(features, features, labels_col, labels_row, margin_arr)
    return out[0, 0]


# speedup vs baseline: 1.0141x; 1.0141x over previous
import functools

import jax
import jax.numpy as jnp
from jax.experimental import pallas as pl
from jax.experimental.pallas import tpu as pltpu

_KP = 64
_INF = float("inf")


def _triplet_block(a_ref, f_ref, lr_ref, lc_ref, m_ref, o_ref, ah_s, b_s, acc,
                   *, nsteps, ba):
    i = pl.program_id(0)
    n, d = f_ref.shape

    @pl.when(i == 0)
    def _():
        f = f_ref[...]
        fsq = jnp.sum(f * f, axis=1, keepdims=True)
        ones = jnp.ones((n, 1), jnp.float32)
        pad = jnp.zeros((n, _KP - (d + 1)), jnp.float32)
        b_s[...] = jnp.concatenate([f, fsq, pad], axis=1)
        ah_s[...] = jnp.concatenate([-2.0 * f, ones, pad], axis=1)

    dims = (((1,), (1,)), ((), ()))
    row = pl.ds(i * ba, ba)
    t = jax.lax.dot_general(ah_s[row, :], b_s[...], dims,
                            preferred_element_type=jnp.float32)

    pos_mask = lr_ref[row, :] == lc_ref[...]
    pos_t = jnp.max(jnp.where(pos_mask, t, -_INF), axis=1, keepdims=True)
    neg_t = jnp.min(jnp.where(pos_mask, _INF, t), axis=1, keepdims=True)
    valid = neg_t < _INF

    a = a_ref[...]
    asq = jnp.sum(a * a, axis=1, keepdims=True)
    pos_d2 = jnp.maximum(pos_t + asq, 0.0)
    neg_d2 = jnp.maximum(jnp.where(valid, neg_t, 0.0) + asq, 0.0)

    margin = m_ref[0, 0]
    per = jnp.maximum(jnp.sqrt(pos_d2) - jnp.sqrt(neg_d2) + margin, 0.0)
    per = jnp.where(valid, per, 0.0)

    s = jnp.sum(per, axis=0, keepdims=True)[0, 0]
    c = jnp.sum(valid.astype(jnp.float32), axis=0, keepdims=True)[0, 0]
    tot_s = jnp.where(i == 0, 0.0, acc[0, 0]) + s
    tot_c = jnp.where(i == 0, 0.0, acc[1, 0]) + c
    acc[0, 0] = tot_s
    acc[1, 0] = tot_c

    @pl.when(i == nsteps - 1)
    def _():
        loss = jnp.where(tot_c > 0.0, tot_s / jnp.maximum(tot_c, 1.0), 0.0)
        o_ref[...] = jnp.full((1, 1), loss, jnp.float32)


def kernel(features, labels, margin):
    n, d = features.shape
    ba = 4096
    nsteps = n // ba
    labels_col = labels.reshape(n, 1).astype(jnp.int32)
    labels_row = labels.reshape(1, n).astype(jnp.int32)
    margin_arr = jnp.asarray(margin, jnp.float32).reshape(1, 1)

    out = pl.pallas_call(
        functools.partial(_triplet_block, nsteps=nsteps, ba=ba),
        grid=(nsteps,),
        in_specs=[
            pl.BlockSpec((ba, d), lambda i: (i, 0)),
            pl.BlockSpec((n, d), lambda i: (0, 0)),
            pl.BlockSpec((n, 1), lambda i: (0, 0)),
            pl.BlockSpec((1, n), lambda i: (0, 0)),
            pl.BlockSpec((1, 1), lambda i: (0, 0)),
        ],
        out_specs=pl.BlockSpec((1, 1), lambda i: (0, 0)),
        out_shape=jax.ShapeDtypeStruct((1, 1), jnp.float32),
        scratch_shapes=[pltpu.VMEM((n, _KP), jnp.float32),
                        pltpu.VMEM((n, _KP), jnp.float32),
                        pltpu.SMEM((2, 1), jnp.float32)],
    )(features, features, labels_col, labels_row, margin_arr)
    return out[0, 0]
